# baseline (speedup 1.0000x reference)
import functools

import jax
import jax.numpy as jnp
from jax import lax
from jax.experimental import pallas as pl
from jax.experimental.pallas import tpu as pltpu
from jax.experimental.pallas import tpu_sc as plsc

BATCH = 4096
SEQ = 200
EMBED_DIM = 64
NUM_CORES = 2
NUM_SUBCORES = 16
NUM_WORKERS = NUM_CORES * NUM_SUBCORES
BTILE = BATCH // NUM_WORKERS
LANES = 16
DHI = EMBED_DIM // 8
BHI = BATCH // 128


def _gather_desc(table_hbm, idx_v, buf, sem, s):
    return pltpu.make_async_copy(table_hbm.at[idx_v.at[s]], buf, sem)


def _wb_descs(obuf, out_hbm, sem, s, bt):
    return [
        pltpu.make_async_copy(obuf.at[pl.ds(8 * g, 8)], out_hbm.at[s, g, bt],
                              sem)
        for g in range(DHI)
    ]


def _transpose_add(obuf, gbuf, pos_v, s):
    @plsc.parallel_loop(0, BTILE, 1, unroll=4)
    def _row(bl):
        for j in range(EMBED_DIM // LANES):
            sl = pl.ds(j * LANES, LANES)
            gbuf[bl, sl] = gbuf[bl, sl] + pos_v[s, sl]

    @plsc.parallel_loop(0, EMBED_DIM, 1, unroll=4)
    def _dim(d):
        col = jnp.full((LANES,), d, jnp.int32)
        for j in range(BTILE // LANES):
            rows = lax.iota(jnp.int32, LANES) + j * LANES
            v = plsc.load_gather(gbuf, [rows, col])
            obuf[d, pl.ds(j * LANES, LANES)] = v


def _tpe_kernel(idxT_hbm, table_hbm, pos_hbm, out_hbm,
                idx_v, pos_v, g_a, g_b, o_a, o_b, gs_a, gs_b, ws_a, ws_b):
    gbufs = [g_a, g_b]
    obufs = [o_a, o_b]
    gsems = [gs_a, gs_b]
    wsems = [ws_a, ws_b]
    wid = lax.axis_index("s") * NUM_CORES + lax.axis_index("c")
    b0 = wid * BTILE

    pltpu.sync_copy(pos_hbm, pos_v)
    pltpu.sync_copy(idxT_hbm.at[:, pl.ds(b0, BTILE)], idx_v)

    for b in range(2):
        _gather_desc(table_hbm, idx_v, gbufs[b], gsems[b], b).start()

    def group(p, carry):
        for b in range(2):
            s = 2 * p + b
            _gather_desc(table_hbm, idx_v, gbufs[b], gsems[b], s).wait()

            @pl.when(s >= 2)
            def _():
                for d in _wb_descs(obufs[b], out_hbm, wsems[b], s - 2, wid):
                    d.wait()

            _transpose_add(obufs[b], gbufs[b], pos_v, s)

            @pl.when(s + 2 < SEQ)
            def _():
                _gather_desc(table_hbm, idx_v, gbufs[b], gsems[b],
                             s + 2).start()

            for d in _wb_descs(obufs[b], out_hbm, wsems[b], s, wid):
                d.start()
        return carry

    lax.fori_loop(0, SEQ // 2, group, 0)

    for b in range(2):
        for d in _wb_descs(obufs[b], out_hbm, wsems[b], SEQ - 2 + b, wid):
            d.wait()


def kernel(inputs, token_table, position_table):
    mesh = plsc.VectorSubcoreMesh(core_axis_name="c", subcore_axis_name="s")
    run = functools.partial(
        pl.kernel,
        out_type=jax.ShapeDtypeStruct((SEQ, DHI, BHI, 8, 128), jnp.float32),
        mesh=mesh,
        scratch_types=(
            [pltpu.VMEM((SEQ, BTILE), jnp.int32),
             pltpu.VMEM((SEQ, EMBED_DIM), jnp.float32)]
            + [pltpu.VMEM((BTILE, EMBED_DIM), jnp.float32) for _ in range(2)]
            + [pltpu.VMEM((EMBED_DIM, 128), jnp.float32) for _ in range(2)]
            + [pltpu.SemaphoreType.DMA for _ in range(4)]
        ),
        compiler_params=pltpu.CompilerParams(
            use_tc_tiling_on_sc=False, needs_layout_passes=False),
    )(_tpe_kernel)
    out5 = run(inputs.T.astype(jnp.int32), token_table, position_table)
    return out5.transpose(2, 4, 0, 1, 3).reshape(BATCH, SEQ, EMBED_DIM)


# speedup vs baseline: 1.6549x; 1.6549x over previous
import functools

import jax
import jax.numpy as jnp
from jax import lax
from jax.experimental import pallas as pl
from jax.experimental.pallas import tpu as pltpu
from jax.experimental.pallas import tpu_sc as plsc

BATCH = 4096
SEQ = 200
EMBED_DIM = 64
NUM_CORES = 2
NUM_SUBCORES = 16
NUM_WORKERS = NUM_CORES * NUM_SUBCORES
BTILE = BATCH // NUM_WORKERS
LANES = 16
DHI = EMBED_DIM // 8
BHI = BATCH // 128


def _gather_desc(table_hbm, idx_v, buf, sem, s):
    return pltpu.make_async_copy(table_hbm.at[idx_v.at[s]], buf, sem)


def _wb_descs(obuf, out_hbm, sem, s, bt):
    return [
        pltpu.make_async_copy(obuf.at[pl.ds(8 * g, 8)], out_hbm.at[s, g, bt],
                              sem)
        for g in range(DHI)
    ]


def _transpose_add(obuf, gbuf, pos_v, s):
    @plsc.parallel_loop(0, BTILE, 1, unroll=4)
    def _row(bl):
        for j in range(EMBED_DIM // LANES):
            sl = pl.ds(j * LANES, LANES)
            gbuf[bl, sl] = gbuf[bl, sl] + pos_v[s, sl]

    @plsc.parallel_loop(0, EMBED_DIM, 1, unroll=4)
    def _diag(c):
        dcol = (lax.iota(jnp.int32, LANES) + c) & (EMBED_DIM - 1)
        for j in range(BTILE // LANES):
            rows = lax.iota(jnp.int32, LANES) + j * LANES
            v = plsc.load_gather(gbuf, [rows, dcol])
            plsc.store_scatter(obuf, [dcol, rows], v)


def _tpe_kernel(idxT_hbm, table_hbm, pos_hbm, out_hbm,
                idx_v, pos_v, g_a, g_b, o_a, o_b, gs_a, gs_b, ws_a, ws_b):
    gbufs = [g_a, g_b]
    obufs = [o_a, o_b]
    gsems = [gs_a, gs_b]
    wsems = [ws_a, ws_b]
    wid = lax.axis_index("s") * NUM_CORES + lax.axis_index("c")
    b0 = wid * BTILE

    pltpu.sync_copy(pos_hbm, pos_v)
    pltpu.sync_copy(idxT_hbm.at[:, pl.ds(b0, BTILE)], idx_v)

    for b in range(2):
        _gather_desc(table_hbm, idx_v, gbufs[b], gsems[b], b).start()

    def group(p, carry):
        for b in range(2):
            s = 2 * p + b
            _gather_desc(table_hbm, idx_v, gbufs[b], gsems[b], s).wait()

            @pl.when(s >= 2)
            def _():
                for d in _wb_descs(obufs[b], out_hbm, wsems[b], s - 2, wid):
                    d.wait()

            _transpose_add(obufs[b], gbufs[b], pos_v, s)

            @pl.when(s + 2 < SEQ)
            def _():
                _gather_desc(table_hbm, idx_v, gbufs[b], gsems[b],
                             s + 2).start()

            for d in _wb_descs(obufs[b], out_hbm, wsems[b], s, wid):
                d.start()
        return carry

    lax.fori_loop(0, SEQ // 2, group, 0)

    for b in range(2):
        for d in _wb_descs(obufs[b], out_hbm, wsems[b], SEQ - 2 + b, wid):
            d.wait()


def kernel(inputs, token_table, position_table):
    mesh = plsc.VectorSubcoreMesh(core_axis_name="c", subcore_axis_name="s")
    run = functools.partial(
        pl.kernel,
        out_type=jax.ShapeDtypeStruct((SEQ, DHI, BHI, 8, 128), jnp.float32),
        mesh=mesh,
        scratch_types=(
            [pltpu.VMEM((SEQ, BTILE), jnp.int32),
             pltpu.VMEM((SEQ, EMBED_DIM), jnp.float32)]
            + [pltpu.VMEM((BTILE, EMBED_DIM), jnp.float32) for _ in range(2)]
            + [pltpu.VMEM((EMBED_DIM, 128), jnp.float32) for _ in range(2)]
            + [pltpu.SemaphoreType.DMA for _ in range(4)]
        ),
        compiler_params=pltpu.CompilerParams(
            use_tc_tiling_on_sc=False, needs_layout_passes=False),
    )(_tpe_kernel)
    out5 = run(inputs.T.astype(jnp.int32), token_table, position_table)
    return out5.transpose(2, 4, 0, 1, 3).reshape(BATCH, SEQ, EMBED_DIM)


# speedup vs baseline: 1.6569x; 1.0013x over previous
import functools

import jax
import jax.numpy as jnp
from jax import lax
from jax.experimental import pallas as pl
from jax.experimental.pallas import tpu as pltpu
from jax.experimental.pallas import tpu_sc as plsc

BATCH = 4096
SEQ = 200
EMBED_DIM = 64
NUM_CORES = 2
NUM_SUBCORES = 16
NUM_WORKERS = NUM_CORES * NUM_SUBCORES
BTILE = BATCH // NUM_WORKERS
LANES = 16
DHI = EMBED_DIM // 8
BHI = BATCH // 128


def _gather_desc(table_hbm, idx_v, buf, sem, s):
    return pltpu.make_async_copy(table_hbm.at[idx_v.at[s]], buf, sem)


def _wb_descs(obuf, out_hbm, sem, s, bt):
    return [
        pltpu.make_async_copy(obuf.at[pl.ds(8 * g, 8)], out_hbm.at[s, g, bt],
                              sem)
        for g in range(DHI)
    ]


def _transpose_add(obuf, gbuf, pos_v, s):
    @plsc.parallel_loop(0, EMBED_DIM, 1, unroll=4)
    def _diag(c):
        dcol = (lax.iota(jnp.int32, LANES) + c) & (EMBED_DIM - 1)
        p = plsc.load_gather(pos_v, [jnp.full((LANES,), s, jnp.int32), dcol])
        for j in range(BTILE // LANES):
            rows = lax.iota(jnp.int32, LANES) + j * LANES
            v = plsc.load_gather(gbuf, [rows, dcol])
            plsc.store_scatter(obuf, [dcol, rows], v + p)


def _tpe_kernel(idxT_hbm, table_hbm, pos_hbm, out_hbm,
                idx_v, pos_v, g_a, g_b, o_a, o_b, gs_a, gs_b, ws_a, ws_b):
    gbufs = [g_a, g_b]
    obufs = [o_a, o_b]
    gsems = [gs_a, gs_b]
    wsems = [ws_a, ws_b]
    wid = lax.axis_index("s") * NUM_CORES + lax.axis_index("c")
    b0 = wid * BTILE

    pltpu.sync_copy(pos_hbm, pos_v)
    pltpu.sync_copy(idxT_hbm.at[:, pl.ds(b0, BTILE)], idx_v)

    for b in range(2):
        _gather_desc(table_hbm, idx_v, gbufs[b], gsems[b], b).start()

    def group(p, carry):
        for b in range(2):
            s = 2 * p + b
            _gather_desc(table_hbm, idx_v, gbufs[b], gsems[b], s).wait()

            @pl.when(s >= 2)
            def _():
                for d in _wb_descs(obufs[b], out_hbm, wsems[b], s - 2, wid):
                    d.wait()

            _transpose_add(obufs[b], gbufs[b], pos_v, s)

            @pl.when(s + 2 < SEQ)
            def _():
                _gather_desc(table_hbm, idx_v, gbufs[b], gsems[b],
                             s + 2).start()

            for d in _wb_descs(obufs[b], out_hbm, wsems[b], s, wid):
                d.start()
        return carry

    lax.fori_loop(0, SEQ // 2, group, 0)

    for b in range(2):
        for d in _wb_descs(obufs[b], out_hbm, wsems[b], SEQ - 2 + b, wid):
            d.wait()


def kernel(inputs, token_table, position_table):
    mesh = plsc.VectorSubcoreMesh(core_axis_name="c", subcore_axis_name="s")
    run = functools.partial(
        pl.kernel,
        out_type=jax.ShapeDtypeStruct((SEQ, DHI, BHI, 8, 128), jnp.float32),
        mesh=mesh,
        scratch_types=(
            [pltpu.VMEM((SEQ, BTILE), jnp.int32),
             pltpu.VMEM((SEQ, EMBED_DIM), jnp.float32)]
            + [pltpu.VMEM((BTILE, EMBED_DIM), jnp.float32) for _ in range(2)]
            + [pltpu.VMEM((EMBED_DIM, 128), jnp.float32) for _ in range(2)]
            + [pltpu.SemaphoreType.DMA for _ in range(4)]
        ),
        compiler_params=pltpu.CompilerParams(
            use_tc_tiling_on_sc=False, needs_layout_passes=False),
    )(_tpe_kernel)
    out5 = run(inputs.T.astype(jnp.int32), token_table, position_table)
    return out5.transpose(2, 4, 0, 1, 3).reshape(BATCH, SEQ, EMBED_DIM)
